# baseline (speedup 1.0000x reference)
import functools

import jax
import jax.numpy as jnp
from jax import lax
from jax.experimental import pallas as pl
from jax.experimental.pallas import tpu as pltpu
from jax.experimental.pallas import tpu_sc as plsc

EMB_DIM = 64
BATCH = 16384
MARGIN = 1.0
EPS = 1e-6

NC = 2
NS = 16
L = 16
NW = NC * NS
B_PER_W = BATCH // NW
CHUNK = 128
N_CHUNKS = B_PER_W // CHUNK
GROUPS = CHUNK // L


def _sc_body(eemb, remb, idx_hbm, out_hbm,
             hidx_v, ridx_v, tidx_v, hbuf, rbuf, tbuf, scores_v, sem):
    wid = lax.axis_index("s") * NC + lax.axis_index("c")

    def chunk_body(tc, _):
        p = tc // N_CHUNKS
        c = tc - p * N_CHUNKS
        pltpu.sync_copy(idx_hbm.at[3 * p + 0, wid, c], hidx_v)
        pltpu.sync_copy(idx_hbm.at[3 * p + 1, wid, c], ridx_v)
        pltpu.sync_copy(idx_hbm.at[3 * p + 2, wid, c], tidx_v)
        ch = pltpu.make_async_copy(eemb.at[hidx_v], hbuf, sem)
        cr = pltpu.make_async_copy(remb.at[ridx_v], rbuf, sem)
        ct = pltpu.make_async_copy(eemb.at[tidx_v], tbuf, sem)
        ch.start()
        cr.start()
        ct.start()
        ch.wait()
        cr.wait()
        ct.wait()

        lane = lax.iota(jnp.int32, L)

        def group_body(g, _):
            acc = jnp.zeros((L,), jnp.float32)
            for rl in range(L):
                row = g * L + rl
                sq = jnp.zeros((L,), jnp.float32)
                for dd in range(EMB_DIM // L):
                    hv = hbuf[row, pl.ds(dd * L, L)]
                    rv = rbuf[row, pl.ds(dd * L, L)]
                    tv = tbuf[row, pl.ds(dd * L, L)]
                    df = hv + rv - tv + EPS
                    sq = sq + df * df
                acc = jnp.where(lane == rl, jnp.sum(sq), acc)
            scores_v[pl.ds(tc * CHUNK + g * L, L)] = acc
            return 0

        lax.fori_loop(0, GROUPS, group_body, 0)
        return 0

    lax.fori_loop(0, 2 * N_CHUNKS, chunk_body, 0)
    pltpu.sync_copy(scores_v, out_hbm.at[wid])


_SC_KERNEL = None


def _get_sc_kernel():
    global _SC_KERNEL
    if _SC_KERNEL is None:
        _SC_KERNEL = pl.kernel(
            _sc_body,
            mesh=plsc.VectorSubcoreMesh(core_axis_name="c", subcore_axis_name="s",
                                        num_cores=NC, num_subcores=NS),
            compiler_params=pltpu.CompilerParams(needs_layout_passes=False,
                                                 use_tc_tiling_on_sc=False),
            out_type=jax.ShapeDtypeStruct((NW, 2 * B_PER_W), jnp.float32),
            scratch_types=[
                pltpu.VMEM((CHUNK,), jnp.int32),
                pltpu.VMEM((CHUNK,), jnp.int32),
                pltpu.VMEM((CHUNK,), jnp.int32),
                pltpu.VMEM((CHUNK, EMB_DIM), jnp.float32),
                pltpu.VMEM((CHUNK, EMB_DIM), jnp.float32),
                pltpu.VMEM((CHUNK, EMB_DIM), jnp.float32),
                pltpu.VMEM((2 * B_PER_W,), jnp.float32),
                pltpu.SemaphoreType.DMA,
            ],
        )
    return _SC_KERNEL


def _loss_body(ps_ref, ns_ref, out_ref):
    ps = jnp.sqrt(ps_ref[...])
    ns = jnp.sqrt(ns_ref[...])
    out_ref[0, 0] = jnp.sum(jnp.maximum(MARGIN + ps - ns, 0.0)) * (1.0 / BATCH)


_loss_kernel = pl.pallas_call(
    _loss_body,
    out_shape=jax.ShapeDtypeStruct((1, 1), jnp.float32),
    out_specs=pl.BlockSpec(memory_space=pltpu.SMEM),
)


def kernel(pos_triples, neg_triples, e_emb, r_emb):
    idx_all = jnp.stack([
        pos_triples[:, 0], pos_triples[:, 1], pos_triples[:, 2],
        neg_triples[:, 0], neg_triples[:, 1], neg_triples[:, 2],
    ]).reshape(6, NW, N_CHUNKS, CHUNK)
    sq = _get_sc_kernel()(e_emb, r_emb, idx_all)
    ps = sq[:, :B_PER_W].reshape(BATCH // CHUNK, CHUNK)
    ns = sq[:, B_PER_W:].reshape(BATCH // CHUNK, CHUNK)
    return _loss_kernel(ps, ns)[0, 0]


# speedup vs baseline: 1.5309x; 1.5309x over previous
import functools

import jax
import jax.numpy as jnp
from jax import lax
from jax.experimental import pallas as pl
from jax.experimental.pallas import tpu as pltpu
from jax.experimental.pallas import tpu_sc as plsc

EMB_DIM = 64
BATCH = 16384
MARGIN = 1.0
EPS = 1e-6

NC = 2
NS = 16
L = 16
NW = NC * NS
B_PER_W = BATCH // NW
CHUNK = 128
N_CHUNKS = B_PER_W // CHUNK
GROUPS = CHUNK // L


def _sc_body(eemb, remb, idx_hbm, out_hbm,
             hidx_v, ridx_v, tidx_v, hbuf, rbuf, tbuf, scores_v, sem):
    wid = lax.axis_index("s") * NC + lax.axis_index("c")

    def chunk_body(tc, _):
        p = tc // N_CHUNKS
        c = tc - p * N_CHUNKS
        pltpu.sync_copy(idx_hbm.at[3 * p + 0, wid, c], hidx_v)
        pltpu.sync_copy(idx_hbm.at[3 * p + 1, wid, c], ridx_v)
        pltpu.sync_copy(idx_hbm.at[3 * p + 2, wid, c], tidx_v)

        def issue_body(g, _):
            hvec = hidx_v[pl.ds(g * L, L)]
            rvec = ridx_v[pl.ds(g * L, L)]
            tvec = tidx_v[pl.ds(g * L, L)]
            for rl in range(L):
                row = g * L + rl
                pltpu.make_async_copy(eemb.at[hvec[rl]], hbuf.at[row], sem).start()
                pltpu.make_async_copy(remb.at[rvec[rl]], rbuf.at[row], sem).start()
                pltpu.make_async_copy(eemb.at[tvec[rl]], tbuf.at[row], sem).start()
            return 0

        lax.fori_loop(0, GROUPS, issue_body, 0)
        pltpu.make_async_copy(eemb.at[pl.ds(0, CHUNK)], hbuf, sem).wait()
        pltpu.make_async_copy(remb.at[pl.ds(0, CHUNK)], rbuf, sem).wait()
        pltpu.make_async_copy(eemb.at[pl.ds(0, CHUNK)], tbuf, sem).wait()

        lane = lax.iota(jnp.int32, L)

        def group_body(g, _):
            acc = jnp.zeros((L,), jnp.float32)
            for rl in range(L):
                row = g * L + rl
                sq = jnp.zeros((L,), jnp.float32)
                for dd in range(EMB_DIM // L):
                    hv = hbuf[row, pl.ds(dd * L, L)]
                    rv = rbuf[row, pl.ds(dd * L, L)]
                    tv = tbuf[row, pl.ds(dd * L, L)]
                    df = hv + rv - tv + EPS
                    sq = sq + df * df
                acc = jnp.where(lane == rl, jnp.sum(sq), acc)
            scores_v[pl.ds(tc * CHUNK + g * L, L)] = acc
            return 0

        lax.fori_loop(0, GROUPS, group_body, 0)
        return 0

    lax.fori_loop(0, 2 * N_CHUNKS, chunk_body, 0)
    pltpu.sync_copy(scores_v, out_hbm.at[wid])


_SC_KERNEL = None


def _get_sc_kernel():
    global _SC_KERNEL
    if _SC_KERNEL is None:
        _SC_KERNEL = pl.kernel(
            _sc_body,
            mesh=plsc.VectorSubcoreMesh(core_axis_name="c", subcore_axis_name="s",
                                        num_cores=NC, num_subcores=NS),
            compiler_params=pltpu.CompilerParams(needs_layout_passes=False,
                                                 use_tc_tiling_on_sc=True),
            out_type=jax.ShapeDtypeStruct((NW, 2 * B_PER_W), jnp.float32),
            scratch_types=[
                pltpu.VMEM((CHUNK,), jnp.int32),
                pltpu.VMEM((CHUNK,), jnp.int32),
                pltpu.VMEM((CHUNK,), jnp.int32),
                pltpu.VMEM((CHUNK, EMB_DIM), jnp.float32),
                pltpu.VMEM((CHUNK, EMB_DIM), jnp.float32),
                pltpu.VMEM((CHUNK, EMB_DIM), jnp.float32),
                pltpu.VMEM((2 * B_PER_W,), jnp.float32),
                pltpu.SemaphoreType.DMA,
            ],
        )
    return _SC_KERNEL


def _loss_body(ps_ref, ns_ref, out_ref):
    ps = jnp.sqrt(ps_ref[...])
    ns = jnp.sqrt(ns_ref[...])
    out_ref[0, 0] = jnp.sum(jnp.maximum(MARGIN + ps - ns, 0.0)) * (1.0 / BATCH)


_loss_kernel = pl.pallas_call(
    _loss_body,
    out_shape=jax.ShapeDtypeStruct((1, 1), jnp.float32),
    out_specs=pl.BlockSpec(memory_space=pltpu.SMEM),
)


def kernel(pos_triples, neg_triples, e_emb, r_emb):
    idx_all = jnp.stack([
        pos_triples[:, 0], pos_triples[:, 1], pos_triples[:, 2],
        neg_triples[:, 0], neg_triples[:, 1], neg_triples[:, 2],
    ]).reshape(6, NW, N_CHUNKS, CHUNK)
    sq = _get_sc_kernel()(e_emb, r_emb, idx_all)
    ps = sq[:, :B_PER_W].reshape(BATCH // CHUNK, CHUNK)
    ns = sq[:, B_PER_W:].reshape(BATCH // CHUNK, CHUNK)
    return _loss_kernel(ps, ns)[0, 0]


# speedup vs baseline: 1.5336x; 1.0017x over previous
import jax
import jax.numpy as jnp
from jax import lax
from jax.experimental import pallas as pl
from jax.experimental.pallas import tpu as pltpu
from jax.experimental.pallas import tpu_sc as plsc

EMB_DIM = 64
BATCH = 16384
MARGIN = 1.0
EPS = 1e-6

NC = 2
NS = 16
L = 16
NW = NC * NS
B_PER_W = BATCH // NW
CHUNK = 128
N_CHUNKS = B_PER_W // CHUNK
GROUPS = CHUNK // L


def _sc_body(etab, rtab, idx_hbm, out_hbm,
             hidx_v, ridx_v, tidx_v, hbuf, rbuf, tbuf, scores_v, sem):
    wid = lax.axis_index("s") * NC + lax.axis_index("c")

    lane = lax.iota(jnp.int32, L)

    def chunk_body(tc, _):
        p = tc // N_CHUNKS
        c = tc - p * N_CHUNKS
        pltpu.sync_copy(idx_hbm.at[3 * p + 0, wid, c], hidx_v)
        pltpu.sync_copy(idx_hbm.at[3 * p + 1, wid, c], ridx_v)
        pltpu.sync_copy(idx_hbm.at[3 * p + 2, wid, c], tidx_v)

        def issue_body(g, _):
            hvec = hidx_v[pl.ds(g * L, L)]
            rvec = ridx_v[pl.ds(g * L, L)]
            tvec = tidx_v[pl.ds(g * L, L)]
            for rl in range(L):
                i = g * L + rl
                pltpu.make_async_copy(etab.at[hvec[rl]], hbuf.at[i], sem).start()
                pltpu.make_async_copy(rtab.at[rvec[rl]], rbuf.at[i], sem).start()
                pltpu.make_async_copy(etab.at[tvec[rl]], tbuf.at[i], sem).start()
            return 0

        lax.fori_loop(0, GROUPS, issue_body, 0)
        pltpu.make_async_copy(etab.at[pl.ds(0, CHUNK)], hbuf, sem).wait()
        pltpu.make_async_copy(rtab.at[pl.ds(0, CHUNK)], rbuf, sem).wait()
        pltpu.make_async_copy(etab.at[pl.ds(0, CHUNK)], tbuf, sem).wait()

        def group_body(g, _):
            acc = jnp.zeros((L,), jnp.float32)
            for rl in range(L):
                row = g * L + rl
                sq = jnp.zeros((L,), jnp.float32)
                for dd in range(EMB_DIM // L):
                    hv = hbuf[row, pl.ds(dd * L, L)]
                    rv = rbuf[row, pl.ds(dd * L, L)]
                    tv = tbuf[row, pl.ds(dd * L, L)]
                    df = hv + rv - tv + EPS
                    sq = sq + df * df
                acc = jnp.where(lane == rl, jnp.sum(sq), acc)
            scores_v[pl.ds(tc * CHUNK + g * L, L)] = acc
            return 0

        lax.fori_loop(0, GROUPS, group_body, 0)
        return 0

    lax.fori_loop(0, 2 * N_CHUNKS, chunk_body, 0)
    pltpu.sync_copy(scores_v, out_hbm.at[wid])


_SC_KERNEL = None


def _get_sc_kernel():
    global _SC_KERNEL
    if _SC_KERNEL is None:
        _SC_KERNEL = pl.kernel(
            _sc_body,
            mesh=plsc.VectorSubcoreMesh(core_axis_name="c", subcore_axis_name="s",
                                        num_cores=NC, num_subcores=NS),
            compiler_params=pltpu.CompilerParams(needs_layout_passes=False,
                                                 use_tc_tiling_on_sc=True),
            out_type=jax.ShapeDtypeStruct((NW, 2 * B_PER_W), jnp.float32),
            scratch_types=[
                pltpu.VMEM((CHUNK,), jnp.int32),
                pltpu.VMEM((CHUNK,), jnp.int32),
                pltpu.VMEM((CHUNK,), jnp.int32),
                pltpu.VMEM((CHUNK, EMB_DIM), jnp.float32),
                pltpu.VMEM((CHUNK, EMB_DIM), jnp.float32),
                pltpu.VMEM((CHUNK, EMB_DIM), jnp.float32),
                pltpu.VMEM((2 * B_PER_W,), jnp.float32),
                pltpu.SemaphoreType.DMA,
            ],
        )
    return _SC_KERNEL


TB = 4096


def _tpose_body(in_ref, out_ref):
    out_ref[...] = in_ref[...].T


_tpose_kernel = pl.pallas_call(
    _tpose_body,
    grid=(pl.cdiv(1000000, TB),),
    in_specs=[pl.BlockSpec((EMB_DIM, TB), lambda j: (0, j))],
    out_specs=pl.BlockSpec((TB, EMB_DIM), lambda j: (j, 0)),
    out_shape=jax.ShapeDtypeStruct((1000000, EMB_DIM), jnp.float32),
)


def _loss_body(ps_ref, ns_ref, out_ref):
    ps = jnp.sqrt(ps_ref[...])
    ns = jnp.sqrt(ns_ref[...])
    out_ref[0, 0] = jnp.sum(jnp.maximum(MARGIN + ps - ns, 0.0)) * (1.0 / BATCH)


_loss_kernel = pl.pallas_call(
    _loss_body,
    out_shape=jax.ShapeDtypeStruct((1, 1), jnp.float32),
    out_specs=pl.BlockSpec(memory_space=pltpu.SMEM),
)


def kernel(pos_triples, neg_triples, e_emb, r_emb):
    idx_all = jnp.stack([
        pos_triples[:, 0], pos_triples[:, 1], pos_triples[:, 2],
        neg_triples[:, 0], neg_triples[:, 1], neg_triples[:, 2],
    ]).reshape(6, NW, N_CHUNKS, CHUNK)
    e_rm = _tpose_kernel(e_emb.T)
    r_rm = _tpose_kernel(r_emb.T)
    sq = _get_sc_kernel()(e_rm, r_rm, idx_all)
    ps = sq[:, :B_PER_W].reshape(BATCH // CHUNK, CHUNK)
    ns = sq[:, B_PER_W:].reshape(BATCH // CHUNK, CHUNK)
    return _loss_kernel(ps, ns)[0, 0]


# speedup vs baseline: 1.8978x; 1.2375x over previous
import jax
import jax.numpy as jnp
from jax import lax
from jax.experimental import pallas as pl
from jax.experimental.pallas import tpu as pltpu
from jax.experimental.pallas import tpu_sc as plsc

EMB_DIM = 64
BATCH = 16384
MARGIN = 1.0
EPS = 1e-6

NC = 2
NS = 16
L = 16
NW = NC * NS
B_PER_W = BATCH // NW
CHUNK = 128
N_CHUNKS = B_PER_W // CHUNK
GROUPS = CHUNK // L


def _sc_body(etab, rtab, idx_hbm, out_hbm,
             hidx_v, ridx_v, tidx_v, hbuf, rbuf, tbuf, scores_v, sem):
    wid = lax.axis_index("s") * NC + lax.axis_index("c")

    lane = lax.iota(jnp.int32, L)

    def chunk_body(tc, _):
        p = tc // N_CHUNKS
        c = tc - p * N_CHUNKS
        pltpu.sync_copy(idx_hbm.at[3 * p + 0, wid, c], hidx_v)
        pltpu.sync_copy(idx_hbm.at[3 * p + 1, wid, c], ridx_v)
        pltpu.sync_copy(idx_hbm.at[3 * p + 2, wid, c], tidx_v)

        def issue_body(g, _):
            hvec = hidx_v[pl.ds(g * L, L)]
            rvec = ridx_v[pl.ds(g * L, L)]
            tvec = tidx_v[pl.ds(g * L, L)]
            for rl in range(L):
                i = g * L + rl
                pltpu.make_async_copy(etab.at[hvec[rl]], hbuf.at[i], sem).start()
                pltpu.make_async_copy(rtab.at[rvec[rl]], rbuf.at[i], sem).start()
                pltpu.make_async_copy(etab.at[tvec[rl]], tbuf.at[i], sem).start()
            return 0

        lax.fori_loop(0, GROUPS, issue_body, 0)
        pltpu.make_async_copy(etab.at[pl.ds(0, CHUNK)], hbuf, sem).wait()
        pltpu.make_async_copy(rtab.at[pl.ds(0, CHUNK)], rbuf, sem).wait()
        pltpu.make_async_copy(etab.at[pl.ds(0, CHUNK)], tbuf, sem).wait()

        def group_body(g, _):
            acc = jnp.zeros((L,), jnp.float32)
            for rl in range(L):
                row = g * L + rl
                sq = jnp.zeros((L,), jnp.float32)
                for dd in range(EMB_DIM // L):
                    hv = hbuf[row, pl.ds(dd * L, L)]
                    rv = rbuf[row, pl.ds(dd * L, L)]
                    tv = tbuf[row, pl.ds(dd * L, L)]
                    df = hv + rv - tv + EPS
                    sq = sq + df * df
                acc = jnp.where(lane == rl, jnp.sum(sq), acc)
            scores_v[pl.ds(tc * CHUNK + g * L, L)] = acc
            return 0

        lax.fori_loop(0, GROUPS, group_body, 0)
        return 0

    lax.fori_loop(0, 2 * N_CHUNKS, chunk_body, 0)
    pltpu.sync_copy(scores_v, out_hbm.at[wid])


_SC_KERNEL = None


def _get_sc_kernel():
    global _SC_KERNEL
    if _SC_KERNEL is None:
        _SC_KERNEL = pl.kernel(
            _sc_body,
            mesh=plsc.VectorSubcoreMesh(core_axis_name="c", subcore_axis_name="s",
                                        num_cores=NC, num_subcores=NS),
            compiler_params=pltpu.CompilerParams(needs_layout_passes=False,
                                                 use_tc_tiling_on_sc=True),
            out_type=jax.ShapeDtypeStruct((NW, 2 * B_PER_W), jnp.float32),
            scratch_types=[
                pltpu.VMEM((CHUNK,), jnp.int32),
                pltpu.VMEM((CHUNK,), jnp.int32),
                pltpu.VMEM((CHUNK,), jnp.int32),
                pltpu.VMEM((CHUNK, EMB_DIM), jnp.float32),
                pltpu.VMEM((CHUNK, EMB_DIM), jnp.float32),
                pltpu.VMEM((CHUNK, EMB_DIM), jnp.float32),
                pltpu.VMEM((2 * B_PER_W,), jnp.float32),
                pltpu.SemaphoreType.DMA,
            ],
        )
    return _SC_KERNEL


TB = 8192


def _tpose_body(in_ref, out_ref):
    out_ref[...] = in_ref[...].T


_tpose_kernel = pl.pallas_call(
    _tpose_body,
    grid=(pl.cdiv(1000000, TB),),
    in_specs=[pl.BlockSpec((EMB_DIM, TB), lambda j: (0, j))],
    out_specs=pl.BlockSpec((TB, EMB_DIM), lambda j: (j, 0)),
    out_shape=jax.ShapeDtypeStruct((1000000, EMB_DIM), jnp.float32),
)


def _loss_body(ps_ref, ns_ref, out_ref):
    ps = jnp.sqrt(ps_ref[...])
    ns = jnp.sqrt(ns_ref[...])
    out_ref[0, 0] = jnp.sum(jnp.maximum(MARGIN + ps - ns, 0.0)) * (1.0 / BATCH)


_loss_kernel = pl.pallas_call(
    _loss_body,
    out_shape=jax.ShapeDtypeStruct((1, 1), jnp.float32),
    out_specs=pl.BlockSpec(memory_space=pltpu.SMEM),
)


def kernel(pos_triples, neg_triples, e_emb, r_emb):
    idx_all = jnp.stack([
        pos_triples[:, 0], pos_triples[:, 1], pos_triples[:, 2],
        neg_triples[:, 0], neg_triples[:, 1], neg_triples[:, 2],
    ]).reshape(6, NW, N_CHUNKS, CHUNK)
    e_rm = _tpose_kernel(e_emb.T)
    r_rm = _tpose_kernel(r_emb.T)
    sq = _get_sc_kernel()(e_rm, r_rm, idx_all)
    ps = sq[:, :B_PER_W].reshape(BATCH // CHUNK, CHUNK)
    ns = sq[:, B_PER_W:].reshape(BATCH // CHUNK, CHUNK)
    return _loss_kernel(ps, ns)[0, 0]


# speedup vs baseline: 2.0367x; 1.0732x over previous
import jax
import jax.numpy as jnp
from jax import lax
from jax.experimental import pallas as pl
from jax.experimental.pallas import tpu as pltpu
from jax.experimental.pallas import tpu_sc as plsc

EMB_DIM = 64
BATCH = 16384
MARGIN = 1.0
EPS = 1e-6

NC = 2
NS = 16
L = 16
NW = NC * NS
B_PER_W = BATCH // NW
CHUNK = 128
N_CHUNKS = B_PER_W // CHUNK
GROUPS = CHUNK // L


def _sc_body(etab, rtab, idx_hbm, out_hbm,
             hidx_v, ridx_v, tidx_v, hbuf, rbuf, tbuf, scores_v, sem):
    wid = lax.axis_index("s") * NC + lax.axis_index("c")

    lane = lax.iota(jnp.int32, L)

    def chunk_body(tc, _):
        p = tc // N_CHUNKS
        c = tc - p * N_CHUNKS
        pltpu.sync_copy(idx_hbm.at[3 * p + 0, wid, c], hidx_v)
        pltpu.sync_copy(idx_hbm.at[3 * p + 1, wid, c], ridx_v)
        pltpu.sync_copy(idx_hbm.at[3 * p + 2, wid, c], tidx_v)

        def issue_body(g, _):
            hvec = hidx_v[pl.ds(g * L, L)]
            rvec = ridx_v[pl.ds(g * L, L)]
            tvec = tidx_v[pl.ds(g * L, L)]
            for rl in range(L):
                i = g * L + rl
                pltpu.make_async_copy(etab.at[hvec[rl]], hbuf.at[i], sem).start()
                pltpu.make_async_copy(rtab.at[rvec[rl]], rbuf.at[i], sem).start()
                pltpu.make_async_copy(etab.at[tvec[rl]], tbuf.at[i], sem).start()
            return 0

        lax.fori_loop(0, GROUPS, issue_body, 0)
        pltpu.make_async_copy(etab.at[pl.ds(0, CHUNK)], hbuf, sem).wait()
        pltpu.make_async_copy(rtab.at[pl.ds(0, CHUNK)], rbuf, sem).wait()
        pltpu.make_async_copy(etab.at[pl.ds(0, CHUNK)], tbuf, sem).wait()

        def group_body(g, _):
            acc = jnp.zeros((L,), jnp.float32)
            for rl in range(L):
                row = g * L + rl
                sq = jnp.zeros((L,), jnp.float32)
                for dd in range(EMB_DIM // L):
                    hv = hbuf[row, pl.ds(dd * L, L)]
                    rv = rbuf[row, pl.ds(dd * L, L)]
                    tv = tbuf[row, pl.ds(dd * L, L)]
                    df = hv + rv - tv + EPS
                    sq = sq + df * df
                acc = jnp.where(lane == rl, jnp.sum(sq), acc)
            scores_v[pl.ds(tc * CHUNK + g * L, L)] = acc
            return 0

        lax.fori_loop(0, GROUPS, group_body, 0)
        return 0

    lax.fori_loop(0, 2 * N_CHUNKS, chunk_body, 0)
    pltpu.sync_copy(scores_v, out_hbm.at[wid])


_SC_KERNEL = None


def _get_sc_kernel():
    global _SC_KERNEL
    if _SC_KERNEL is None:
        _SC_KERNEL = pl.kernel(
            _sc_body,
            mesh=plsc.VectorSubcoreMesh(core_axis_name="c", subcore_axis_name="s",
                                        num_cores=NC, num_subcores=NS),
            compiler_params=pltpu.CompilerParams(needs_layout_passes=False,
                                                 use_tc_tiling_on_sc=True),
            out_type=jax.ShapeDtypeStruct((NW, 2 * B_PER_W), jnp.float32),
            scratch_types=[
                pltpu.VMEM((CHUNK,), jnp.int32),
                pltpu.VMEM((CHUNK,), jnp.int32),
                pltpu.VMEM((CHUNK,), jnp.int32),
                pltpu.VMEM((CHUNK, EMB_DIM), jnp.float32),
                pltpu.VMEM((CHUNK, EMB_DIM), jnp.float32),
                pltpu.VMEM((CHUNK, EMB_DIM), jnp.float32),
                pltpu.VMEM((2 * B_PER_W,), jnp.float32),
                pltpu.SemaphoreType.DMA,
            ],
        )
    return _SC_KERNEL


TB = 16384


def _tpose_body(in_ref, out_ref):
    out_ref[...] = in_ref[...].T


_tpose_kernel = pl.pallas_call(
    _tpose_body,
    grid=(pl.cdiv(1000000, TB),),
    in_specs=[pl.BlockSpec((EMB_DIM, TB), lambda j: (0, j))],
    out_specs=pl.BlockSpec((TB, EMB_DIM), lambda j: (j, 0)),
    out_shape=jax.ShapeDtypeStruct((1000000, EMB_DIM), jnp.float32),
)


def _loss_body(ps_ref, ns_ref, out_ref):
    ps = jnp.sqrt(ps_ref[...])
    ns = jnp.sqrt(ns_ref[...])
    out_ref[0, 0] = jnp.sum(jnp.maximum(MARGIN + ps - ns, 0.0)) * (1.0 / BATCH)


_loss_kernel = pl.pallas_call(
    _loss_body,
    out_shape=jax.ShapeDtypeStruct((1, 1), jnp.float32),
    out_specs=pl.BlockSpec(memory_space=pltpu.SMEM),
)


def kernel(pos_triples, neg_triples, e_emb, r_emb):
    idx_all = jnp.stack([
        pos_triples[:, 0], pos_triples[:, 1], pos_triples[:, 2],
        neg_triples[:, 0], neg_triples[:, 1], neg_triples[:, 2],
    ]).reshape(6, NW, N_CHUNKS, CHUNK)
    e_rm = _tpose_kernel(e_emb.T)
    r_rm = _tpose_kernel(r_emb.T)
    sq = _get_sc_kernel()(e_rm, r_rm, idx_all)
    ps = sq[:, :B_PER_W].reshape(BATCH // CHUNK, CHUNK)
    ns = sq[:, B_PER_W:].reshape(BATCH // CHUNK, CHUNK)
    return _loss_kernel(ps, ns)[0, 0]


# speedup vs baseline: 2.0887x; 1.0255x over previous
import jax
import jax.numpy as jnp
from jax import lax
from jax.experimental import pallas as pl
from jax.experimental.pallas import tpu as pltpu
from jax.experimental.pallas import tpu_sc as plsc

EMB_DIM = 64
BATCH = 16384
MARGIN = 1.0
EPS = 1e-6

NC = 2
NS = 16
L = 16
NW = NC * NS
B_PER_W = BATCH // NW
CHUNK = 128
N_CHUNKS = B_PER_W // CHUNK
GROUPS = CHUNK // L


def _sc_body(etab, rtab, idx_hbm, out_hbm,
             hidx_v, ridx_v, tidx_v, hbuf, rbuf, tbuf, scores_v, sem):
    wid = lax.axis_index("s") * NC + lax.axis_index("c")

    lane = lax.iota(jnp.int32, L)

    def chunk_body(tc, _):
        p = tc // N_CHUNKS
        c = tc - p * N_CHUNKS
        pltpu.sync_copy(idx_hbm.at[3 * p + 0, wid, c], hidx_v)
        pltpu.sync_copy(idx_hbm.at[3 * p + 1, wid, c], ridx_v)
        pltpu.sync_copy(idx_hbm.at[3 * p + 2, wid, c], tidx_v)

        def issue_body(g, _):
            hvec = hidx_v[pl.ds(g * L, L)]
            rvec = ridx_v[pl.ds(g * L, L)]
            tvec = tidx_v[pl.ds(g * L, L)]
            for rl in range(L):
                i = g * L + rl
                pltpu.make_async_copy(etab.at[hvec[rl]], hbuf.at[i], sem).start()
                pltpu.make_async_copy(rtab.at[rvec[rl]], rbuf.at[i], sem).start()
                pltpu.make_async_copy(etab.at[tvec[rl]], tbuf.at[i], sem).start()
            return 0

        lax.fori_loop(0, GROUPS, issue_body, 0)
        pltpu.make_async_copy(etab.at[pl.ds(0, CHUNK)], hbuf, sem).wait()
        pltpu.make_async_copy(rtab.at[pl.ds(0, CHUNK)], rbuf, sem).wait()
        pltpu.make_async_copy(etab.at[pl.ds(0, CHUNK)], tbuf, sem).wait()

        def group_body(g, _):
            acc = jnp.zeros((L,), jnp.float32)
            for rl in range(L):
                row = g * L + rl
                sq = jnp.zeros((L,), jnp.float32)
                for dd in range(EMB_DIM // L):
                    hv = hbuf[row, pl.ds(dd * L, L)]
                    rv = rbuf[row, pl.ds(dd * L, L)]
                    tv = tbuf[row, pl.ds(dd * L, L)]
                    df = hv + rv - tv + EPS
                    sq = sq + df * df
                acc = jnp.where(lane == rl, jnp.sum(sq), acc)
            scores_v[pl.ds(tc * CHUNK + g * L, L)] = acc
            return 0

        lax.fori_loop(0, GROUPS, group_body, 0)
        return 0

    lax.fori_loop(0, 2 * N_CHUNKS, chunk_body, 0)
    pltpu.sync_copy(scores_v, out_hbm.at[wid])


_SC_KERNEL = None


def _get_sc_kernel():
    global _SC_KERNEL
    if _SC_KERNEL is None:
        _SC_KERNEL = pl.kernel(
            _sc_body,
            mesh=plsc.VectorSubcoreMesh(core_axis_name="c", subcore_axis_name="s",
                                        num_cores=NC, num_subcores=NS),
            compiler_params=pltpu.CompilerParams(needs_layout_passes=False,
                                                 use_tc_tiling_on_sc=True),
            out_type=jax.ShapeDtypeStruct((NW, 2 * B_PER_W), jnp.float32),
            scratch_types=[
                pltpu.VMEM((CHUNK,), jnp.int32),
                pltpu.VMEM((CHUNK,), jnp.int32),
                pltpu.VMEM((CHUNK,), jnp.int32),
                pltpu.VMEM((CHUNK, EMB_DIM), jnp.float32),
                pltpu.VMEM((CHUNK, EMB_DIM), jnp.float32),
                pltpu.VMEM((CHUNK, EMB_DIM), jnp.float32),
                pltpu.VMEM((2 * B_PER_W,), jnp.float32),
                pltpu.SemaphoreType.DMA,
            ],
        )
    return _SC_KERNEL


TB = 32768


def _tpose_body(in_ref, out_ref):
    out_ref[...] = in_ref[...].T


_tpose_kernel = pl.pallas_call(
    _tpose_body,
    grid=(pl.cdiv(1000000, TB),),
    in_specs=[pl.BlockSpec((EMB_DIM, TB), lambda j: (0, j))],
    out_specs=pl.BlockSpec((TB, EMB_DIM), lambda j: (j, 0)),
    out_shape=jax.ShapeDtypeStruct((1000000, EMB_DIM), jnp.float32),
)


def _loss_body(ps_ref, ns_ref, out_ref):
    ps = jnp.sqrt(ps_ref[...])
    ns = jnp.sqrt(ns_ref[...])
    out_ref[0, 0] = jnp.sum(jnp.maximum(MARGIN + ps - ns, 0.0)) * (1.0 / BATCH)


_loss_kernel = pl.pallas_call(
    _loss_body,
    out_shape=jax.ShapeDtypeStruct((1, 1), jnp.float32),
    out_specs=pl.BlockSpec(memory_space=pltpu.SMEM),
)


def kernel(pos_triples, neg_triples, e_emb, r_emb):
    idx_all = jnp.stack([
        pos_triples[:, 0], pos_triples[:, 1], pos_triples[:, 2],
        neg_triples[:, 0], neg_triples[:, 1], neg_triples[:, 2],
    ]).reshape(6, NW, N_CHUNKS, CHUNK)
    e_rm = _tpose_kernel(e_emb.T)
    r_rm = _tpose_kernel(r_emb.T)
    sq = _get_sc_kernel()(e_rm, r_rm, idx_all)
    ps = sq[:, :B_PER_W].reshape(BATCH // CHUNK, CHUNK)
    ns = sq[:, B_PER_W:].reshape(BATCH // CHUNK, CHUNK)
    return _loss_kernel(ps, ns)[0, 0]


# speedup vs baseline: 2.2721x; 1.0878x over previous
import jax
import jax.numpy as jnp
from jax import lax
from jax.experimental import pallas as pl
from jax.experimental.pallas import tpu as pltpu
from jax.experimental.pallas import tpu_sc as plsc

EMB_DIM = 64
BATCH = 16384
MARGIN = 1.0
EPS = 1e-6

NC = 2
NS = 16
L = 16
NW = NC * NS
B_PER_W = BATCH // NW
CHUNK = 128
N_CHUNKS = B_PER_W // CHUNK
GROUPS = CHUNK // L


HALF = 524288


def _sc_body(etab, rtab, idx_hbm, out_hbm,
             hidx_v, ridx_v, tidx_v, hbuf, rbuf, tbuf, scores_v, sem):
    wid = lax.axis_index("s") * NC + lax.axis_index("c")

    lane = lax.iota(jnp.int32, L)

    def chunk_body(tc, _):
        p = tc // N_CHUNKS
        c = tc - p * N_CHUNKS
        pltpu.sync_copy(idx_hbm.at[3 * p + 0, wid, c], hidx_v)
        pltpu.sync_copy(idx_hbm.at[3 * p + 1, wid, c], ridx_v)
        pltpu.sync_copy(idx_hbm.at[3 * p + 2, wid, c], tidx_v)

        def issue_body(g, _):
            hvec = hidx_v[pl.ds(g * L, L)]
            rvec = ridx_v[pl.ds(g * L, L)]
            tvec = tidx_v[pl.ds(g * L, L)]
            for rl in range(L):
                i = g * L + rl
                hp = hvec[rl] - jnp.where(hvec[rl] >= HALF, HALF, 0)
                rp = rvec[rl] - jnp.where(rvec[rl] >= HALF, HALF, 0)
                tp = tvec[rl] - jnp.where(tvec[rl] >= HALF, HALF, 0)
                pltpu.make_async_copy(etab.at[hp], hbuf.at[i], sem).start()
                pltpu.make_async_copy(rtab.at[rp], rbuf.at[i], sem).start()
                pltpu.make_async_copy(etab.at[tp], tbuf.at[i], sem).start()
            return 0

        lax.fori_loop(0, GROUPS, issue_body, 0)
        pltpu.make_async_copy(etab.at[pl.ds(0, CHUNK)], hbuf, sem).wait()
        pltpu.make_async_copy(rtab.at[pl.ds(0, CHUNK)], rbuf, sem).wait()
        pltpu.make_async_copy(etab.at[pl.ds(0, CHUNK)], tbuf, sem).wait()

        def group_body(g, _):
            hvec = hidx_v[pl.ds(g * L, L)]
            rvec = ridx_v[pl.ds(g * L, L)]
            tvec = tidx_v[pl.ds(g * L, L)]
            acc = jnp.zeros((L,), jnp.float32)
            for rl in range(L):
                row = g * L + rl
                ho = jnp.where(hvec[rl] >= HALF, EMB_DIM, 0)
                ro = jnp.where(rvec[rl] >= HALF, EMB_DIM, 0)
                to = jnp.where(tvec[rl] >= HALF, EMB_DIM, 0)
                sq = jnp.zeros((L,), jnp.float32)
                for dd in range(EMB_DIM // L):
                    hv = hbuf[row, pl.ds(ho + dd * L, L)]
                    rv = rbuf[row, pl.ds(ro + dd * L, L)]
                    tv = tbuf[row, pl.ds(to + dd * L, L)]
                    df = hv + rv - tv + EPS
                    sq = sq + df * df
                acc = jnp.where(lane == rl, jnp.sum(sq), acc)
            scores_v[pl.ds(tc * CHUNK + g * L, L)] = acc
            return 0

        lax.fori_loop(0, GROUPS, group_body, 0)
        return 0

    lax.fori_loop(0, 2 * N_CHUNKS, chunk_body, 0)
    pltpu.sync_copy(scores_v, out_hbm.at[wid])


_SC_KERNEL = None


def _get_sc_kernel():
    global _SC_KERNEL
    if _SC_KERNEL is None:
        _SC_KERNEL = pl.kernel(
            _sc_body,
            mesh=plsc.VectorSubcoreMesh(core_axis_name="c", subcore_axis_name="s",
                                        num_cores=NC, num_subcores=NS),
            compiler_params=pltpu.CompilerParams(needs_layout_passes=False,
                                                 use_tc_tiling_on_sc=True),
            out_type=jax.ShapeDtypeStruct((NW, 2 * B_PER_W), jnp.float32),
            scratch_types=[
                pltpu.VMEM((CHUNK,), jnp.int32),
                pltpu.VMEM((CHUNK,), jnp.int32),
                pltpu.VMEM((CHUNK,), jnp.int32),
                pltpu.VMEM((CHUNK, 2 * EMB_DIM), jnp.float32),
                pltpu.VMEM((CHUNK, 2 * EMB_DIM), jnp.float32),
                pltpu.VMEM((CHUNK, 2 * EMB_DIM), jnp.float32),
                pltpu.VMEM((2 * B_PER_W,), jnp.float32),
                pltpu.SemaphoreType.DMA,
            ],
        )
    return _SC_KERNEL


TB = 16384


def _tpose_body(a_ref, b_ref, out_ref):
    out_ref[:, 0:EMB_DIM] = a_ref[...].T
    out_ref[:, EMB_DIM:2 * EMB_DIM] = b_ref[...].T


_tpose_kernel = pl.pallas_call(
    _tpose_body,
    grid=(HALF // TB,),
    in_specs=[pl.BlockSpec((EMB_DIM, TB), lambda j: (0, j)),
              pl.BlockSpec((EMB_DIM, TB),
                           lambda j: (0, jnp.minimum(j + HALF // TB,
                                                     1000000 // TB)))],
    out_specs=pl.BlockSpec((TB, 2 * EMB_DIM), lambda j: (j, 0)),
    out_shape=jax.ShapeDtypeStruct((HALF, 2 * EMB_DIM), jnp.float32),
)


def _loss_body(ps_ref, ns_ref, out_ref):
    ps = jnp.sqrt(ps_ref[...])
    ns = jnp.sqrt(ns_ref[...])
    out_ref[0, 0] = jnp.sum(jnp.maximum(MARGIN + ps - ns, 0.0)) * (1.0 / BATCH)


_loss_kernel = pl.pallas_call(
    _loss_body,
    out_shape=jax.ShapeDtypeStruct((1, 1), jnp.float32),
    out_specs=pl.BlockSpec(memory_space=pltpu.SMEM),
)


def kernel(pos_triples, neg_triples, e_emb, r_emb):
    idx_all = jnp.stack([
        pos_triples[:, 0], pos_triples[:, 1], pos_triples[:, 2],
        neg_triples[:, 0], neg_triples[:, 1], neg_triples[:, 2],
    ]).reshape(6, NW, N_CHUNKS, CHUNK)
    e_rm = _tpose_kernel(e_emb.T, e_emb.T)
    r_rm = _tpose_kernel(r_emb.T, r_emb.T)
    sq = _get_sc_kernel()(e_rm, r_rm, idx_all)
    ps = sq[:, :B_PER_W].reshape(BATCH // CHUNK, CHUNK)
    ns = sq[:, B_PER_W:].reshape(BATCH // CHUNK, CHUNK)
    return _loss_kernel(ps, ns)[0, 0]
